# baseline (speedup 1.0000x reference)
import functools

import jax
import jax.numpy as jnp
from jax import lax
from jax.experimental import pallas as pl
from jax.experimental.pallas import tpu as pltpu
from jax.experimental.pallas import tpu_sc as plsc

N0 = 10000
N1 = 2000
E0 = 320000
E1 = 32000
D = 128

NTILES = 32
N0A = 10240
N1A = 2048
E0P = 327680
E1P = 32768
F32 = jnp.float32
I32 = jnp.int32

_MESH = dict(core_axis_name="c", subcore_axis_name="s",
             num_cores=2, num_subcores=16)


def _seg_pass(n_acc, ept, cw, nchunks, table_w=D):
  slice_rows = n_acc // 16
  assert ept == cw * nchunks and slice_rows % 128 == 0

  pieces = []
  r = 0
  while r < slice_rows:
    pieces.append((r, min(cw, slice_rows - r)))
    r += pieces[-1][1]

  blocked = (cw == 128)
  cpb = min(nchunks, 16)
  assert not blocked or nchunks % cpb == 0

  if blocked:
    idx_scr = [pltpu.VMEM((cpb, 128), I32) for _ in range(4)]
  else:
    idx_scr = [pltpu.VMEM((cw,), I32) for _ in range(4)]

  def body(table, src, dst, zrows, acc_out, g0, g1, sidx0, sidx1,
           didx0, didx1, acc_sh, sem0, sem1, ssem0, ssem1):
    grows = (g0, g1)
    sidx = (sidx0, sidx1)
    didx = (didx0, didx1)
    sem = (sem0, sem1)
    ssem = (ssem0, ssem1)
    c = lax.axis_index("c")
    s = lax.axis_index("s")
    wid = s * 2 + c
    base_r = s * slice_rows

    pltpu.sync_copy(zrows.at[pl.ds(0, cw), :], g0)
    for off, rows in pieces:
      pltpu.sync_copy(g0.at[pl.ds(0, rows), :],
                      acc_sh.at[pl.ds(base_r + off, rows), :])
    plsc.subcore_barrier()

    ebase = wid * ept

    if blocked:
      rows_per_tile = ept // 128

      def load_block(blk):
        p = blk % 2
        rb = wid * rows_per_tile + blk * cpb
        pltpu.sync_copy(src.at[pl.ds(rb, cpb), :], sidx[p])
        pltpu.sync_copy(dst.at[pl.ds(rb, cpb), :], didx[p])

      def fire(k):
        b, p, r = k % 2, (k // cpb) % 2, k % cpb
        pltpu.async_copy(table.at[sidx[p].at[r]], grows[b], sem[b])

      def wait_gather(k):
        b, p, r = k % 2, (k // cpb) % 2, k % cpb
        pltpu.make_async_copy(table.at[sidx[p].at[r]], grows[b],
                              sem[b]).wait()

      def scat(k, action):
        b, p, r = k % 2, (k // cpb) % 2, k % cpb
        desc = pltpu.make_async_copy(grows[b], acc_sh.at[didx[p].at[r]],
                                     ssem[b])
        if action == "start":
          desc.start(add=True)
        else:
          desc.wait()

      load_block(0)
      fire(0)
      for k in range(nchunks):
        if k >= 1:
          scat(k - 1, "wait")
        if k + 1 < nchunks:
          if (k + 1) % cpb == 0:
            load_block((k + 1) // cpb)
          fire(k + 1)
        wait_gather(k)
        scat(k, "start")
      scat(nchunks - 1, "wait")
    else:
      def load_and_fire(k):
        b = k % 2
        off = ebase + k * cw
        pltpu.sync_copy(src.at[pl.ds(off, cw)], sidx[b])
        pltpu.sync_copy(dst.at[pl.ds(off, cw)], didx[b])
        pltpu.async_copy(table.at[sidx[b]], grows[b], sem[b])

      def drain_and_scatter(k):
        b = k % 2
        pltpu.make_async_copy(table.at[sidx[b]], grows[b], sem[b]).wait()
        pltpu.sync_copy(grows[b], acc_sh.at[didx[b]], add=True)

      load_and_fire(0)
      for k in range(nchunks):
        if k + 1 < nchunks:
          load_and_fire(k + 1)
        drain_and_scatter(k)

    plsc.subcore_barrier()

    for off, rows in pieces:
      pltpu.sync_copy(acc_sh.at[pl.ds(base_r + off, rows), :],
                      g0.at[pl.ds(0, rows), :])
      pltpu.sync_copy(g0.at[pl.ds(0, rows), :],
                      acc_out.at[c, pl.ds(base_r + off, rows), :])

  return pl.kernel(
      body,
      out_type=jax.ShapeDtypeStruct((2, n_acc, table_w), F32),
      mesh=plsc.VectorSubcoreMesh(**_MESH),
      scratch_types=[
          pltpu.VMEM((cw, table_w), F32),
          pltpu.VMEM((cw, table_w), F32),
      ] + idx_scr + [
          pltpu.VMEM_SHARED((n_acc, table_w), F32),
          pltpu.SemaphoreType.DMA,
          pltpu.SemaphoreType.DMA,
          pltpu.SemaphoreType.DMA,
          pltpu.SemaphoreType.DMA,
      ],
  )


_H_SPECS = ((N0A, E0P // NTILES), (N1A, N0A // NTILES), (N1A, E1P // NTILES))


def _hist3_pass():
  def body(d0, d1, d2, zeros0, zeros1, o0, o1, o2,
           didx, dl0, dl1, dl2, redbuf, out1d, sh0, sh1, sh2):
    c = lax.axis_index("c")
    s = lax.axis_index("s")
    wid = s * 2 + c
    ones16 = jnp.ones((16,), F32)

    for dst, zeros1d, degloc, deg_sh, (n_acc, ept) in (
        (d0, zeros0, dl0, sh0, _H_SPECS[0]),
        (d1, zeros1, dl1, sh1, _H_SPECS[1]),
        (d2, zeros1, dl2, sh2, _H_SPECS[2]),
    ):
      pltpu.sync_copy(zeros1d, degloc)
      pltpu.sync_copy(dst.at[pl.ds(wid * ept, ept)], didx.at[pl.ds(0, ept)])

      def sub(j, c2):
        idx = didx[pl.ds(j * 16, 16)]
        plsc.addupdate_scatter(degloc, [idx], ones16)
        return c2

      lax.fori_loop(0, ept // 16, sub, 0)
      pltpu.sync_copy(degloc, deg_sh.at[s])
    plsc.subcore_barrier()

    for deg_out, deg_sh, (n_acc, ept) in (
        (o0, sh0, _H_SPECS[0]), (o1, sh1, _H_SPECS[1]), (o2, sh2, _H_SPECS[2])
    ):
      sr = n_acc // 16
      for p in range(sr // 128):
        colbase = s * sr + p * 128
        pltpu.sync_copy(deg_sh.at[:, pl.ds(colbase, 128)], redbuf)
        for g in range(8):
          tot = redbuf[0, pl.ds(g * 16, 16)]
          for r in range(1, 16):
            tot = tot + redbuf[r, pl.ds(g * 16, 16)]
          out1d[pl.ds(p * 128 + g * 16, 16)] = tot
      pltpu.sync_copy(out1d.at[pl.ds(0, sr)], deg_out.at[c, pl.ds(s * sr, sr)])

  return pl.kernel(
      body,
      out_type=(jax.ShapeDtypeStruct((2, N0A), F32),
                jax.ShapeDtypeStruct((2, N1A), F32),
                jax.ShapeDtypeStruct((2, N1A), F32)),
      mesh=plsc.VectorSubcoreMesh(**_MESH),
      compiler_params=pltpu.CompilerParams(needs_layout_passes=False),
      scratch_types=[
          pltpu.VMEM((E0P // NTILES,), I32),
          pltpu.VMEM((N0A,), F32),
          pltpu.VMEM((N1A,), F32),
          pltpu.VMEM((N1A,), F32),
          pltpu.VMEM((16, 128), F32),
          pltpu.VMEM((N0A // 16,), F32),
          pltpu.VMEM_SHARED((16, N0A), F32),
          pltpu.VMEM_SHARED((16, N1A), F32),
          pltpu.VMEM_SHARED((16, N1A), F32),
      ],
  )


def _gather_pass(n_out, table_rows, table_w, cw, nchunks):
  ept = n_out // NTILES
  assert ept == cw * nchunks

  def body(table, idx, out, grows, ibuf, sem):
    c = lax.axis_index("c")
    s = lax.axis_index("s")
    wid = s * 2 + c
    base = wid * ept

    def chunk(k, carry):
      off = base + k * cw
      pltpu.sync_copy(idx.at[pl.ds(off, cw)], ibuf)
      pltpu.async_copy(table.at[ibuf], grows, sem).wait()
      pltpu.sync_copy(grows, out.at[pl.ds(off, cw), :])
      return carry

    lax.fori_loop(0, nchunks, chunk, 0)

  return pl.kernel(
      body,
      out_type=jax.ShapeDtypeStruct((n_out, table_w), F32),
      mesh=plsc.VectorSubcoreMesh(**_MESH),
      scratch_types=[
          pltpu.VMEM((cw, table_w), F32),
          pltpu.VMEM((cw,), I32),
          pltpu.SemaphoreType.DMA,
      ],
  )


_RB = 2048


def _m0(x, wn, ws, b):
  def body(x_ref, wn_ref, ws_ref, b_ref, u_ref, s_ref):
    xb = x_ref[...]
    u_ref[...] = jnp.dot(xb, wn_ref[...], preferred_element_type=F32)
    s_ref[...] = jnp.dot(xb, ws_ref[...], preferred_element_type=F32) + b_ref[...]

  return pl.pallas_call(
      body,
      grid=(N0A // _RB,),
      in_specs=[
          pl.BlockSpec((_RB, D), lambda i: (i, 0)),
          pl.BlockSpec((D, D), lambda i: (0, 0)),
          pl.BlockSpec((D, D), lambda i: (0, 0)),
          pl.BlockSpec((D,), lambda i: (0,)),
      ],
      out_specs=[
          pl.BlockSpec((_RB, D), lambda i: (i, 0)),
          pl.BlockSpec((_RB, D), lambda i: (i, 0)),
      ],
      out_shape=[
          jax.ShapeDtypeStruct((N0A, D), F32),
          jax.ShapeDtypeStruct((N0A, D), F32),
      ],
  )(x, wn, ws, b)


def _rdeg(deg_ref):
  return 1.0 / jnp.maximum(deg_ref[...], 1.0)


def _m1(s0, acc, deg):
  def body(s_ref, a_ref, d_ref, h_ref):
    h_ref[...] = s_ref[...] + (a_ref[0] + a_ref[1]) * _rdeg(d_ref)

  return pl.pallas_call(
      body,
      grid=(N0A // _RB,),
      in_specs=[
          pl.BlockSpec((_RB, D), lambda i: (i, 0)),
          pl.BlockSpec((2, _RB, D), lambda i: (0, i, 0)),
          pl.BlockSpec((_RB, 1), lambda i: (i, 0)),
      ],
      out_specs=pl.BlockSpec((_RB, D), lambda i: (i, 0)),
      out_shape=jax.ShapeDtypeStruct((N0A, D), F32),
  )(s0, acc, deg)


def _m2(acc, pdeg, wn, ws, b):
  def body(a_ref, d_ref, wn_ref, ws_ref, b_ref, u_ref, s_ref):
    h1a = jnp.maximum((a_ref[0] + a_ref[1]) * _rdeg(d_ref), 0.0)
    u_ref[...] = jnp.dot(h1a, wn_ref[...], preferred_element_type=F32)
    s_ref[...] = jnp.dot(h1a, ws_ref[...], preferred_element_type=F32) + b_ref[...]

  return pl.pallas_call(
      body,
      out_shape=[
          jax.ShapeDtypeStruct((N1A, D), F32),
          jax.ShapeDtypeStruct((N1A, D), F32),
      ],
  )(acc, pdeg, wn, ws, b)


def _m3(s1, acc, deg, wn2t, ws2t, b2):
  def body(s_ref, a_ref, d_ref, wn_ref, ws_ref, b_ref, vw_ref):
    h1 = jnp.maximum(s_ref[...] + (a_ref[0] + a_ref[1]) * _rdeg(d_ref), 0.0)
    vw_ref[:, :D] = jnp.dot(h1, wn_ref[...], preferred_element_type=F32)
    vw_ref[:, D:] = jnp.dot(h1, ws_ref[...], preferred_element_type=F32) + b_ref[...]

  return pl.pallas_call(
      body,
      out_shape=jax.ShapeDtypeStruct((N1A, 2 * D), F32),
  )(s1, acc, deg, wn2t, ws2t, b2)


def _m4(g, h0, wn2b, ws2b):
  def body(g_ref, h_ref, wn_ref, ws_ref, u_ref, s_ref):
    hb = h_ref[...]
    u_ref[...] = g_ref[:, :D] + jnp.dot(hb, wn_ref[...], preferred_element_type=F32)
    s_ref[...] = g_ref[:, D:] + jnp.dot(hb, ws_ref[...], preferred_element_type=F32)

  return pl.pallas_call(
      body,
      grid=(N0A // _RB,),
      in_specs=[
          pl.BlockSpec((_RB, 2 * D), lambda i: (i, 0)),
          pl.BlockSpec((_RB, D), lambda i: (i, 0)),
          pl.BlockSpec((D, D), lambda i: (0, 0)),
          pl.BlockSpec((D, D), lambda i: (0, 0)),
      ],
      out_specs=[
          pl.BlockSpec((_RB, D), lambda i: (i, 0)),
          pl.BlockSpec((_RB, D), lambda i: (i, 0)),
      ],
      out_shape=[
          jax.ShapeDtypeStruct((N0A, D), F32),
          jax.ShapeDtypeStruct((N0A, D), F32),
      ],
  )(g, h0, wn2b, ws2b)


def _m5(s2, acc, deg, wm, bm):
  def body(s_ref, a_ref, d_ref, wm_ref, bm_ref, z_ref):
    i = pl.program_id(0)
    h = jnp.maximum(s_ref[...] + (a_ref[0] + a_ref[1]) * _rdeg(d_ref), 0.0)
    z = jnp.dot(h, wm_ref[...], preferred_element_type=F32) + bm_ref[0]
    rows = i * _RB + lax.broadcasted_iota(I32, (_RB, 1), 0)
    z_ref[...] = jnp.where(rows < N0, z, -1e30)

  return pl.pallas_call(
      body,
      grid=(N0A // _RB,),
      in_specs=[
          pl.BlockSpec((_RB, D), lambda i: (i, 0)),
          pl.BlockSpec((2, _RB, D), lambda i: (0, i, 0)),
          pl.BlockSpec((_RB, 1), lambda i: (i, 0)),
          pl.BlockSpec((D, 1), lambda i: (0, 0)),
          pl.BlockSpec((1,), lambda i: (0,)),
      ],
      out_specs=pl.BlockSpec((_RB, 1), lambda i: (i, 0)),
      out_shape=jax.ShapeDtypeStruct((N0A, 1), F32),
  )(s2, acc, deg, wm, bm)


def _m6(z):
  def body(z_ref, o_ref):
    zv = z_ref[...]
    e = jnp.exp(zv - jnp.max(zv))
    o_ref[...] = e / jnp.sum(e)

  return pl.pallas_call(
      body,
      out_shape=jax.ShapeDtypeStruct((N0A // 128, 128), F32),
  )(z)


def kernel(x, e0_src, e0_dst, e1_src, e1_dst, down_dst,
           w_self0, w_neigh0, b0, w_self1, w_neigh1, b1,
           w_self2, w_neigh2, b2, w_mlp, b_mlp):
  p0 = E0P - E0
  pad0s = ((jnp.arange(p0, dtype=I32) * 13) % N0A)
  pad0d = N0 + (jnp.arange(p0, dtype=I32) % (N0A - N0))
  e0s = jnp.concatenate([e0_src.astype(I32), pad0s])
  e0d = jnp.concatenate([e0_dst.astype(I32), pad0d])
  p1 = E1P - E1
  pad1s = ((jnp.arange(p1, dtype=I32) * 13) % N1A)
  pad1d = N1 + (jnp.arange(p1, dtype=I32) % (N1A - N1))
  e1s = jnp.concatenate([e1_src.astype(I32), pad1s])
  e1d = jnp.concatenate([e1_dst.astype(I32), pad1d])
  pd = N0A - N0
  ddst = jnp.concatenate(
      [down_dst.astype(I32), N1 + (jnp.arange(pd, dtype=I32) % (N1A - N1))])
  idx0 = jnp.arange(N0A, dtype=I32)
  xp = jnp.pad(x, ((0, N0A - N0), (0, 0)))

  z128 = jnp.zeros((128, D), F32)
  z0a = jnp.zeros((N0A,), F32)
  z1a = jnp.zeros((N1A,), F32)

  deg0, pdeg, deg1 = _hist3_pass()(e0d, ddst, e1d, z0a, z1a)
  deg0c = (deg0[0] + deg0[1])[:, None]
  pdegc = (pdeg[0] + pdeg[1])[:, None]
  deg1c = (deg1[0] + deg1[1])[:, None]

  e0s2 = e0s.reshape(E0P // 128, 128)
  e0d2 = e0d.reshape(E0P // 128, 128)
  e1s2 = e1s.reshape(E1P // 128, 128)
  e1d2 = e1d.reshape(E1P // 128, 128)

  u0, s0 = _m0(xp, w_neigh0, w_self0, b0)
  acc0 = _seg_pass(N0A, 10240, 128, 80)(u0, e0s2, e0d2, z128)
  h0 = _m1(s0, acc0, deg0c)

  accd = _seg_pass(N1A, 320, 80, 4)(h0, idx0, ddst, z128)
  u1, s1 = _m2(accd, pdegc, w_neigh1, w_self1, b1)

  acc1 = _seg_pass(N1A, 1024, 128, 8)(u1, e1s2, e1d2, z128)
  vw = _m3(s1, acc1, deg1c, w_neigh2[:D], w_self2[:D], b2)

  g = _gather_pass(N0A, N1A, 2 * D, 80, 4)(vw, ddst)
  u2, s2 = _m4(g, h0, w_neigh2[D:], w_self2[D:])
  acc2 = _seg_pass(N0A, 10240, 128, 80)(u2, e0s2, e0d2, z128)

  z = _m5(s2, acc2, deg0c, w_mlp, b_mlp)
  sm = _m6(z.reshape(N0A // 128, 128))
  return sm.reshape(N0A, 1)[:N0]


# speedup vs baseline: 1.0019x; 1.0019x over previous
import jax
import jax.numpy as jnp
from jax import lax
from jax.experimental import pallas as pl
from jax.experimental.pallas import tpu as pltpu
from jax.experimental.pallas import tpu_sc as plsc

N0 = 10000
N1 = 2000
E0 = 320000
E1 = 32000
D = 128

NTILES = 32
N0A = 10240
N1A = 2048
E0P = 327680
E1P = 32768
F32 = jnp.float32
I32 = jnp.int32

_MESH = dict(core_axis_name="c", subcore_axis_name="s",
             num_cores=2, num_subcores=16)


def _seg_pass(n_acc, ept, cw, nchunks, table_w=D, dtype=F32):
  slice_rows = n_acc // 16
  assert ept == cw * nchunks and slice_rows % 128 == 0

  pieces = []
  r = 0
  while r < slice_rows:
    pieces.append((r, min(cw, slice_rows - r)))
    r += pieces[-1][1]

  blocked = (cw == 128)
  cpb = min(nchunks, 16)
  assert not blocked or nchunks % cpb == 0

  if blocked:
    idx_scr = [pltpu.VMEM((cpb, 128), I32) for _ in range(4)]
  else:
    idx_scr = [pltpu.VMEM((cw,), I32) for _ in range(4)]

  def body(table, src, dst, zrows, acc_out, g0, g1, sidx0, sidx1,
           didx0, didx1, acc_sh, sem0, sem1, ssem0, ssem1):
    grows = (g0, g1)
    sidx = (sidx0, sidx1)
    didx = (didx0, didx1)
    sem = (sem0, sem1)
    ssem = (ssem0, ssem1)
    c = lax.axis_index("c")
    s = lax.axis_index("s")
    wid = s * 2 + c
    base_r = s * slice_rows

    pltpu.sync_copy(zrows.at[pl.ds(0, cw), :], g0)
    for off, rows in pieces:
      pltpu.sync_copy(g0.at[pl.ds(0, rows), :],
                      acc_sh.at[pl.ds(base_r + off, rows), :])
    plsc.subcore_barrier()

    ebase = wid * ept

    if blocked:
      rows_per_tile = ept // 128

      def load_block(blk):
        p = blk % 2
        rb = wid * rows_per_tile + blk * cpb
        pltpu.sync_copy(src.at[pl.ds(rb, cpb), :], sidx[p])
        pltpu.sync_copy(dst.at[pl.ds(rb, cpb), :], didx[p])

      def fire(k):
        b, p, r = k % 2, (k // cpb) % 2, k % cpb
        pltpu.async_copy(table.at[sidx[p].at[r]], grows[b], sem[b])

      def wait_gather(k):
        b, p, r = k % 2, (k // cpb) % 2, k % cpb
        pltpu.make_async_copy(table.at[sidx[p].at[r]], grows[b],
                              sem[b]).wait()

      def scat(k, action):
        b, p, r = k % 2, (k // cpb) % 2, k % cpb
        desc = pltpu.make_async_copy(grows[b], acc_sh.at[didx[p].at[r]],
                                     ssem[b])
        if action == "start":
          desc.start(add=True)
        else:
          desc.wait()

      load_block(0)
      fire(0)
      for k in range(nchunks):
        if k >= 1:
          scat(k - 1, "wait")
        if k + 1 < nchunks:
          if (k + 1) % cpb == 0:
            load_block((k + 1) // cpb)
          fire(k + 1)
        wait_gather(k)
        scat(k, "start")
      scat(nchunks - 1, "wait")
    else:
      def load_and_fire(k):
        b = k % 2
        off = ebase + k * cw
        pltpu.sync_copy(src.at[pl.ds(off, cw)], sidx[b])
        pltpu.sync_copy(dst.at[pl.ds(off, cw)], didx[b])
        pltpu.async_copy(table.at[sidx[b]], grows[b], sem[b])

      def drain_and_scatter(k):
        b = k % 2
        pltpu.make_async_copy(table.at[sidx[b]], grows[b], sem[b]).wait()
        pltpu.sync_copy(grows[b], acc_sh.at[didx[b]], add=True)

      load_and_fire(0)
      for k in range(nchunks):
        if k + 1 < nchunks:
          load_and_fire(k + 1)
        drain_and_scatter(k)

    plsc.subcore_barrier()

    for off, rows in pieces:
      pltpu.sync_copy(acc_sh.at[pl.ds(base_r + off, rows), :],
                      g0.at[pl.ds(0, rows), :])
      pltpu.sync_copy(g0.at[pl.ds(0, rows), :],
                      acc_out.at[c, pl.ds(base_r + off, rows), :])

  return pl.kernel(
      body,
      out_type=jax.ShapeDtypeStruct((2, n_acc, table_w), dtype),
      mesh=plsc.VectorSubcoreMesh(**_MESH),
      scratch_types=[
          pltpu.VMEM((cw, table_w), dtype),
          pltpu.VMEM((cw, table_w), dtype),
      ] + idx_scr + [
          pltpu.VMEM_SHARED((n_acc, table_w), dtype),
          pltpu.SemaphoreType.DMA,
          pltpu.SemaphoreType.DMA,
          pltpu.SemaphoreType.DMA,
          pltpu.SemaphoreType.DMA,
      ],
  )


_H_SPECS = ((N0A, E0P // NTILES), (N1A, N0A // NTILES), (N1A, E1P // NTILES))


def _hist3_pass():
  def body(d0, d1, d2, zeros0, zeros1, o0, o1, o2,
           didx, dl0, dl1, dl2, redbuf, out1d, sh0, sh1, sh2):
    c = lax.axis_index("c")
    s = lax.axis_index("s")
    wid = s * 2 + c
    ones16 = jnp.ones((16,), F32)

    for dst, zeros1d, degloc, deg_sh, (n_acc, ept) in (
        (d0, zeros0, dl0, sh0, _H_SPECS[0]),
        (d1, zeros1, dl1, sh1, _H_SPECS[1]),
        (d2, zeros1, dl2, sh2, _H_SPECS[2]),
    ):
      pltpu.sync_copy(zeros1d, degloc)
      pltpu.sync_copy(dst.at[pl.ds(wid * ept, ept)], didx.at[pl.ds(0, ept)])

      def sub(j, c2):
        idx = didx[pl.ds(j * 16, 16)]
        plsc.addupdate_scatter(degloc, [idx], ones16)
        return c2

      lax.fori_loop(0, ept // 16, sub, 0)
      pltpu.sync_copy(degloc, deg_sh.at[s])
    plsc.subcore_barrier()

    for deg_out, deg_sh, (n_acc, ept) in (
        (o0, sh0, _H_SPECS[0]), (o1, sh1, _H_SPECS[1]), (o2, sh2, _H_SPECS[2])
    ):
      sr = n_acc // 16
      for p in range(sr // 128):
        colbase = s * sr + p * 128
        pltpu.sync_copy(deg_sh.at[:, pl.ds(colbase, 128)], redbuf)
        for g in range(8):
          tot = redbuf[0, pl.ds(g * 16, 16)]
          for r in range(1, 16):
            tot = tot + redbuf[r, pl.ds(g * 16, 16)]
          out1d[pl.ds(p * 128 + g * 16, 16)] = tot
      pltpu.sync_copy(out1d.at[pl.ds(0, sr)], deg_out.at[c, pl.ds(s * sr, sr)])

  return pl.kernel(
      body,
      out_type=(jax.ShapeDtypeStruct((2, N0A), F32),
                jax.ShapeDtypeStruct((2, N1A), F32),
                jax.ShapeDtypeStruct((2, N1A), F32)),
      mesh=plsc.VectorSubcoreMesh(**_MESH),
      compiler_params=pltpu.CompilerParams(needs_layout_passes=False),
      scratch_types=[
          pltpu.VMEM((E0P // NTILES,), I32),
          pltpu.VMEM((N0A,), F32),
          pltpu.VMEM((N1A,), F32),
          pltpu.VMEM((N1A,), F32),
          pltpu.VMEM((16, 128), F32),
          pltpu.VMEM((N0A // 16,), F32),
          pltpu.VMEM_SHARED((16, N0A), F32),
          pltpu.VMEM_SHARED((16, N1A), F32),
          pltpu.VMEM_SHARED((16, N1A), F32),
      ],
  )


def _gather_pass(n_out, table_rows, table_w, cw, nchunks):
  ept = n_out // NTILES
  assert ept == cw * nchunks

  def body(table, idx, out, grows, ibuf, sem):
    c = lax.axis_index("c")
    s = lax.axis_index("s")
    wid = s * 2 + c
    base = wid * ept

    def chunk(k, carry):
      off = base + k * cw
      pltpu.sync_copy(idx.at[pl.ds(off, cw)], ibuf)
      pltpu.async_copy(table.at[ibuf], grows, sem).wait()
      pltpu.sync_copy(grows, out.at[pl.ds(off, cw), :])
      return carry

    lax.fori_loop(0, nchunks, chunk, 0)

  return pl.kernel(
      body,
      out_type=jax.ShapeDtypeStruct((n_out, table_w), F32),
      mesh=plsc.VectorSubcoreMesh(**_MESH),
      scratch_types=[
          pltpu.VMEM((cw, table_w), F32),
          pltpu.VMEM((cw,), I32),
          pltpu.SemaphoreType.DMA,
      ],
  )


_RB = 2048


def _m0(x, wn, ws, b):
  def body(x_ref, wn_ref, ws_ref, b_ref, u_ref, s_ref):
    xb = x_ref[...]
    u_ref[...] = jnp.dot(xb, wn_ref[...], preferred_element_type=F32)
    s_ref[...] = jnp.dot(xb, ws_ref[...], preferred_element_type=F32) + b_ref[...]

  return pl.pallas_call(
      body,
      grid=(N0A // _RB,),
      in_specs=[
          pl.BlockSpec((_RB, D), lambda i: (i, 0)),
          pl.BlockSpec((D, D), lambda i: (0, 0)),
          pl.BlockSpec((D, D), lambda i: (0, 0)),
          pl.BlockSpec((D,), lambda i: (0,)),
      ],
      out_specs=[
          pl.BlockSpec((_RB, D), lambda i: (i, 0)),
          pl.BlockSpec((_RB, D), lambda i: (i, 0)),
      ],
      out_shape=[
          jax.ShapeDtypeStruct((N0A, D), F32),
          jax.ShapeDtypeStruct((N0A, D), F32),
      ],
  )(x, wn, ws, b)


def _rdeg(deg_ref):
  return 1.0 / jnp.maximum(deg_ref[...], 1.0)


def _m1(s0, acc, deg):
  def body(s_ref, a_ref, d_ref, h_ref):
    h_ref[...] = s_ref[...] + (a_ref[0] + a_ref[1]) * _rdeg(d_ref)

  return pl.pallas_call(
      body,
      grid=(N0A // _RB,),
      in_specs=[
          pl.BlockSpec((_RB, D), lambda i: (i, 0)),
          pl.BlockSpec((2, _RB, D), lambda i: (0, i, 0)),
          pl.BlockSpec((_RB, 1), lambda i: (i, 0)),
      ],
      out_specs=pl.BlockSpec((_RB, D), lambda i: (i, 0)),
      out_shape=jax.ShapeDtypeStruct((N0A, D), F32),
  )(s0, acc, deg)


def _m2(acc, pdeg, wn, ws, b):
  def body(a_ref, d_ref, wn_ref, ws_ref, b_ref, u_ref, s_ref):
    h1a = jnp.maximum((a_ref[0] + a_ref[1]) * _rdeg(d_ref), 0.0)
    u_ref[...] = jnp.dot(h1a, wn_ref[...], preferred_element_type=F32)
    s_ref[...] = jnp.dot(h1a, ws_ref[...], preferred_element_type=F32) + b_ref[...]

  return pl.pallas_call(
      body,
      out_shape=[
          jax.ShapeDtypeStruct((N1A, D), F32),
          jax.ShapeDtypeStruct((N1A, D), F32),
      ],
  )(acc, pdeg, wn, ws, b)


def _m3(s1, acc, deg, wn2t, ws2t, b2):
  def body(s_ref, a_ref, d_ref, wn_ref, ws_ref, b_ref, vw_ref):
    h1 = jnp.maximum(s_ref[...] + (a_ref[0] + a_ref[1]) * _rdeg(d_ref), 0.0)
    vw_ref[:, :D] = jnp.dot(h1, wn_ref[...], preferred_element_type=F32)
    vw_ref[:, D:] = jnp.dot(h1, ws_ref[...], preferred_element_type=F32) + b_ref[...]

  return pl.pallas_call(
      body,
      out_shape=jax.ShapeDtypeStruct((N1A, 2 * D), F32),
  )(s1, acc, deg, wn2t, ws2t, b2)


def _m4(g, h0, wn2b, ws2b):
  def body(g_ref, h_ref, wn_ref, ws_ref, u_ref, s_ref):
    hb = h_ref[...]
    u_ref[...] = g_ref[:, :D] + jnp.dot(hb, wn_ref[...],
                                        preferred_element_type=F32)
    s_ref[...] = g_ref[:, D:] + jnp.dot(hb, ws_ref[...], preferred_element_type=F32)

  return pl.pallas_call(
      body,
      grid=(N0A // _RB,),
      in_specs=[
          pl.BlockSpec((_RB, 2 * D), lambda i: (i, 0)),
          pl.BlockSpec((_RB, D), lambda i: (i, 0)),
          pl.BlockSpec((D, D), lambda i: (0, 0)),
          pl.BlockSpec((D, D), lambda i: (0, 0)),
      ],
      out_specs=[
          pl.BlockSpec((_RB, D), lambda i: (i, 0)),
          pl.BlockSpec((_RB, D), lambda i: (i, 0)),
      ],
      out_shape=[
          jax.ShapeDtypeStruct((N0A, D), F32),
          jax.ShapeDtypeStruct((N0A, D), F32),
      ],
  )(g, h0, wn2b, ws2b)


def _m5(s2, acc, deg, wm, bm):
  def body(s_ref, a_ref, d_ref, wm_ref, bm_ref, z_ref):
    i = pl.program_id(0)
    h = jnp.maximum(s_ref[...] + (a_ref[0] + a_ref[1]) * _rdeg(d_ref), 0.0)
    z = jnp.dot(h, wm_ref[...], preferred_element_type=F32) + bm_ref[0]
    rows = i * _RB + lax.broadcasted_iota(I32, (_RB, 1), 0)
    z_ref[...] = jnp.where(rows < N0, z, -1e30)

  return pl.pallas_call(
      body,
      grid=(N0A // _RB,),
      in_specs=[
          pl.BlockSpec((_RB, D), lambda i: (i, 0)),
          pl.BlockSpec((2, _RB, D), lambda i: (0, i, 0)),
          pl.BlockSpec((_RB, 1), lambda i: (i, 0)),
          pl.BlockSpec((D, 1), lambda i: (0, 0)),
          pl.BlockSpec((1,), lambda i: (0,)),
      ],
      out_specs=pl.BlockSpec((_RB, 1), lambda i: (i, 0)),
      out_shape=jax.ShapeDtypeStruct((N0A, 1), F32),
  )(s2, acc, deg, wm, bm)


def _m6(z):
  def body(z_ref, o_ref):
    zv = z_ref[...]
    e = jnp.exp(zv - jnp.max(zv))
    o_ref[...] = e / jnp.sum(e)

  return pl.pallas_call(
      body,
      out_shape=jax.ShapeDtypeStruct((N0A // 128, 128), F32),
  )(z)


def kernel(x, e0_src, e0_dst, e1_src, e1_dst, down_dst,
           w_self0, w_neigh0, b0, w_self1, w_neigh1, b1,
           w_self2, w_neigh2, b2, w_mlp, b_mlp):
  p0 = E0P - E0
  pad0s = ((jnp.arange(p0, dtype=I32) * 13) % N0A)
  pad0d = N0 + (jnp.arange(p0, dtype=I32) % (N0A - N0))
  e0s = jnp.concatenate([e0_src.astype(I32), pad0s])
  e0d = jnp.concatenate([e0_dst.astype(I32), pad0d])
  p1 = E1P - E1
  pad1s = ((jnp.arange(p1, dtype=I32) * 13) % N1A)
  pad1d = N1 + (jnp.arange(p1, dtype=I32) % (N1A - N1))
  e1s = jnp.concatenate([e1_src.astype(I32), pad1s])
  e1d = jnp.concatenate([e1_dst.astype(I32), pad1d])
  pd = N0A - N0
  ddst = jnp.concatenate(
      [down_dst.astype(I32), N1 + (jnp.arange(pd, dtype=I32) % (N1A - N1))])
  idx0 = jnp.arange(N0A, dtype=I32)
  xp = jnp.pad(x, ((0, N0A - N0), (0, 0)))

  z128 = jnp.zeros((128, D), F32)
  z0a = jnp.zeros((N0A,), F32)
  z1a = jnp.zeros((N1A,), F32)

  deg0, pdeg, deg1 = _hist3_pass()(e0d, ddst, e1d, z0a, z1a)
  deg0c = (deg0[0] + deg0[1])[:, None]
  pdegc = (pdeg[0] + pdeg[1])[:, None]
  deg1c = (deg1[0] + deg1[1])[:, None]

  e0s2 = e0s.reshape(E0P // 128, 128)
  e0d2 = e0d.reshape(E0P // 128, 128)
  e1s2 = e1s.reshape(E1P // 128, 128)
  e1d2 = e1d.reshape(E1P // 128, 128)

  u0, s0 = _m0(xp, w_neigh0, w_self0, b0)
  acc0 = _seg_pass(N0A, 10240, 128, 80)(u0, e0s2, e0d2, z128)
  h0 = _m1(s0, acc0, deg0c)

  accd = _seg_pass(N1A, 320, 80, 4)(h0, idx0, ddst, z128)
  u1, s1 = _m2(accd, pdegc, w_neigh1, w_self1, b1)

  acc1 = _seg_pass(N1A, 1024, 128, 8)(u1, e1s2, e1d2, z128)
  vw = _m3(s1, acc1, deg1c, w_neigh2[:D], w_self2[:D], b2)

  g = _gather_pass(N0A, N1A, 2 * D, 80, 4)(vw, ddst)
  u2, s2 = _m4(g, h0, w_neigh2[D:], w_self2[D:])
  acc2 = _seg_pass(N0A, 10240, 128, 80)(u2, e0s2, e0d2, z128)

  z = _m5(s2, acc2, deg0c, w_mlp, b_mlp)
  sm = _m6(z.reshape(N0A // 128, 128))
  return sm.reshape(N0A, 1)[:N0]


# speedup vs baseline: 1.0030x; 1.0011x over previous
import jax
import jax.numpy as jnp
from jax import lax
from jax.experimental import pallas as pl
from jax.experimental.pallas import tpu as pltpu
from jax.experimental.pallas import tpu_sc as plsc

N0 = 10000
N1 = 2000
E0 = 320000
E1 = 32000
D = 128

NTILES = 32
N0A = 10240
N1A = 2048
E0P = 327680
E1P = 32768
F32 = jnp.float32
I32 = jnp.int32

_MESH = dict(core_axis_name="c", subcore_axis_name="s",
             num_cores=2, num_subcores=16)


def _seg_pass(n_acc, ept, cw, nchunks, table_w=D):
  slice_rows = n_acc // 16
  assert ept == cw * nchunks and slice_rows % 128 == 0

  pieces = []
  r = 0
  while r < slice_rows:
    pieces.append((r, min(cw, slice_rows - r)))
    r += pieces[-1][1]

  blocked = (cw == 128)
  cpb = min(nchunks, 16)
  assert not blocked or nchunks % cpb == 0

  if blocked:
    idx_scr = [pltpu.VMEM((cpb, 128), I32) for _ in range(4)]
  else:
    idx_scr = [pltpu.VMEM((cw,), I32) for _ in range(4)]

  def body(table, src, dst, zrows, acc_out, g0, g1, sidx0, sidx1,
           didx0, didx1, acc_sh, sem0, sem1, ssem0, ssem1):
    grows = (g0, g1)
    sidx = (sidx0, sidx1)
    didx = (didx0, didx1)
    sem = (sem0, sem1)
    ssem = (ssem0, ssem1)
    c = lax.axis_index("c")
    s = lax.axis_index("s")
    wid = s * 2 + c
    base_r = s * slice_rows

    pltpu.sync_copy(zrows.at[pl.ds(0, cw), :], g0)
    for off, rows in pieces:
      pltpu.sync_copy(g0.at[pl.ds(0, rows), :],
                      acc_sh.at[pl.ds(base_r + off, rows), :])
    plsc.subcore_barrier()

    ebase = wid * ept

    if blocked:
      rows_per_tile = ept // 128

      def load_block(blk):
        p = blk % 2
        rb = wid * rows_per_tile + blk * cpb
        pltpu.sync_copy(src.at[pl.ds(rb, cpb), :], sidx[p])
        pltpu.sync_copy(dst.at[pl.ds(rb, cpb), :], didx[p])

      def fire(k):
        b, p, r = k % 2, (k // cpb) % 2, k % cpb
        pltpu.async_copy(table.at[sidx[p].at[r]], grows[b], sem[b])

      def wait_gather(k):
        b, p, r = k % 2, (k // cpb) % 2, k % cpb
        pltpu.make_async_copy(table.at[sidx[p].at[r]], grows[b],
                              sem[b]).wait()

      def scat(k, action):
        b, p, r = k % 2, (k // cpb) % 2, k % cpb
        desc = pltpu.make_async_copy(grows[b], acc_sh.at[didx[p].at[r]],
                                     ssem[b])
        if action == "start":
          desc.start(add=True)
        else:
          desc.wait()

      load_block(0)
      fire(0)
      for k in range(nchunks):
        if k >= 1:
          scat(k - 1, "wait")
        if k + 1 < nchunks:
          if (k + 1) % cpb == 0:
            load_block((k + 1) // cpb)
          fire(k + 1)
        wait_gather(k)
        scat(k, "start")
      scat(nchunks - 1, "wait")
    else:
      def load_and_fire(k):
        b = k % 2
        off = ebase + k * cw
        pltpu.sync_copy(src.at[pl.ds(off, cw)], sidx[b])
        pltpu.sync_copy(dst.at[pl.ds(off, cw)], didx[b])
        pltpu.async_copy(table.at[sidx[b]], grows[b], sem[b])

      def drain_and_scatter(k):
        b = k % 2
        pltpu.make_async_copy(table.at[sidx[b]], grows[b], sem[b]).wait()
        pltpu.sync_copy(grows[b], acc_sh.at[didx[b]], add=True)

      load_and_fire(0)
      for k in range(nchunks):
        if k + 1 < nchunks:
          load_and_fire(k + 1)
        drain_and_scatter(k)

    plsc.subcore_barrier()

    for off, rows in pieces:
      pltpu.sync_copy(acc_sh.at[pl.ds(base_r + off, rows), :],
                      g0.at[pl.ds(0, rows), :])
      pltpu.sync_copy(g0.at[pl.ds(0, rows), :],
                      acc_out.at[c, pl.ds(base_r + off, rows), :])

  return pl.kernel(
      body,
      out_type=jax.ShapeDtypeStruct((2, n_acc, table_w), F32),
      mesh=plsc.VectorSubcoreMesh(**_MESH),
      scratch_types=[
          pltpu.VMEM((cw, table_w), F32),
          pltpu.VMEM((cw, table_w), F32),
      ] + idx_scr + [
          pltpu.VMEM_SHARED((n_acc, table_w), F32),
          pltpu.SemaphoreType.DMA,
          pltpu.SemaphoreType.DMA,
          pltpu.SemaphoreType.DMA,
          pltpu.SemaphoreType.DMA,
      ],
  )


_H_SPECS = ((N0A, E0P // NTILES), (N1A, N0A // NTILES), (N1A, E1P // NTILES))


def _hist3_pass():
  def body(d0, d1, d2, zeros0, zeros1, o0, o1, o2,
           didx, dl0, dl1, dl2, redbuf, out1d, sh0, sh1, sh2):
    c = lax.axis_index("c")
    s = lax.axis_index("s")
    wid = s * 2 + c
    ones16 = jnp.ones((16,), F32)

    for dst, zeros1d, degloc, deg_sh, (n_acc, ept) in (
        (d0, zeros0, dl0, sh0, _H_SPECS[0]),
        (d1, zeros1, dl1, sh1, _H_SPECS[1]),
        (d2, zeros1, dl2, sh2, _H_SPECS[2]),
    ):
      pltpu.sync_copy(zeros1d, degloc)
      pltpu.sync_copy(dst.at[pl.ds(wid * ept, ept)], didx.at[pl.ds(0, ept)])

      def sub(j, c2):
        idx = didx[pl.ds(j * 16, 16)]
        plsc.addupdate_scatter(degloc, [idx], ones16)
        return c2

      lax.fori_loop(0, ept // 16, sub, 0)
      pltpu.sync_copy(degloc, deg_sh.at[s])
    plsc.subcore_barrier()

    for deg_out, deg_sh, (n_acc, ept) in (
        (o0, sh0, _H_SPECS[0]), (o1, sh1, _H_SPECS[1]), (o2, sh2, _H_SPECS[2])
    ):
      sr = n_acc // 16
      for p in range(sr // 128):
        colbase = s * sr + p * 128
        pltpu.sync_copy(deg_sh.at[:, pl.ds(colbase, 128)], redbuf)
        for g in range(8):
          tot = redbuf[0, pl.ds(g * 16, 16)]
          for r in range(1, 16):
            tot = tot + redbuf[r, pl.ds(g * 16, 16)]
          out1d[pl.ds(p * 128 + g * 16, 16)] = tot
      pltpu.sync_copy(out1d.at[pl.ds(0, sr)], deg_out.at[c, pl.ds(s * sr, sr)])

  return pl.kernel(
      body,
      out_type=(jax.ShapeDtypeStruct((2, N0A), F32),
                jax.ShapeDtypeStruct((2, N1A), F32),
                jax.ShapeDtypeStruct((2, N1A), F32)),
      mesh=plsc.VectorSubcoreMesh(**_MESH),
      compiler_params=pltpu.CompilerParams(needs_layout_passes=False),
      scratch_types=[
          pltpu.VMEM((E0P // NTILES,), I32),
          pltpu.VMEM((N0A,), F32),
          pltpu.VMEM((N1A,), F32),
          pltpu.VMEM((N1A,), F32),
          pltpu.VMEM((16, 128), F32),
          pltpu.VMEM((N0A // 16,), F32),
          pltpu.VMEM_SHARED((16, N0A), F32),
          pltpu.VMEM_SHARED((16, N1A), F32),
          pltpu.VMEM_SHARED((16, N1A), F32),
      ],
  )


def _gather_pass(n_out, table_rows, table_w, cw, nchunks):
  ept = n_out // NTILES
  assert ept == cw * nchunks

  def body(table, idx, out, grows, ibuf, sem):
    c = lax.axis_index("c")
    s = lax.axis_index("s")
    wid = s * 2 + c
    base = wid * ept

    def chunk(k, carry):
      off = base + k * cw
      pltpu.sync_copy(idx.at[pl.ds(off, cw)], ibuf)
      pltpu.async_copy(table.at[ibuf], grows, sem).wait()
      pltpu.sync_copy(grows, out.at[pl.ds(off, cw), :])
      return carry

    lax.fori_loop(0, nchunks, chunk, 0)

  return pl.kernel(
      body,
      out_type=jax.ShapeDtypeStruct((n_out, table_w), F32),
      mesh=plsc.VectorSubcoreMesh(**_MESH),
      scratch_types=[
          pltpu.VMEM((cw, table_w), F32),
          pltpu.VMEM((cw,), I32),
          pltpu.SemaphoreType.DMA,
      ],
  )


_RB = 2048


def _m0(x, wn, ws, b):
  def body(x_ref, wn_ref, ws_ref, b_ref, u_ref, s_ref):
    xb = x_ref[...]
    u_ref[...] = jnp.dot(xb, wn_ref[...], preferred_element_type=F32)
    s_ref[...] = jnp.dot(xb, ws_ref[...], preferred_element_type=F32) + b_ref[...]

  return pl.pallas_call(
      body,
      grid=(N0A // _RB,),
      in_specs=[
          pl.BlockSpec((_RB, D), lambda i: (i, 0)),
          pl.BlockSpec((D, D), lambda i: (0, 0)),
          pl.BlockSpec((D, D), lambda i: (0, 0)),
          pl.BlockSpec((D,), lambda i: (0,)),
      ],
      out_specs=[
          pl.BlockSpec((_RB, D), lambda i: (i, 0)),
          pl.BlockSpec((_RB, D), lambda i: (i, 0)),
      ],
      out_shape=[
          jax.ShapeDtypeStruct((N0A, D), F32),
          jax.ShapeDtypeStruct((N0A, D), F32),
      ],
  )(x, wn, ws, b)


def _rdeg(deg_ref):
  return 1.0 / jnp.maximum(deg_ref[...], 1.0)


def _m1(s0, acc, deg):
  def body(s_ref, a_ref, d_ref, h_ref):
    h_ref[...] = s_ref[...] + (a_ref[0] + a_ref[1]) * _rdeg(d_ref)

  return pl.pallas_call(
      body,
      grid=(N0A // _RB,),
      in_specs=[
          pl.BlockSpec((_RB, D), lambda i: (i, 0)),
          pl.BlockSpec((2, _RB, D), lambda i: (0, i, 0)),
          pl.BlockSpec((_RB, 1), lambda i: (i, 0)),
      ],
      out_specs=pl.BlockSpec((_RB, D), lambda i: (i, 0)),
      out_shape=jax.ShapeDtypeStruct((N0A, D), F32),
  )(s0, acc, deg)


def _m2(acc, pdeg, wn, ws, b):
  def body(a_ref, d_ref, wn_ref, ws_ref, b_ref, u_ref, s_ref):
    h1a = jnp.maximum((a_ref[0] + a_ref[1]) * _rdeg(d_ref), 0.0)
    u_ref[...] = jnp.dot(h1a, wn_ref[...], preferred_element_type=F32)
    s_ref[...] = jnp.dot(h1a, ws_ref[...], preferred_element_type=F32) + b_ref[...]

  return pl.pallas_call(
      body,
      out_shape=[
          jax.ShapeDtypeStruct((N1A, D), F32),
          jax.ShapeDtypeStruct((N1A, D), F32),
      ],
  )(acc, pdeg, wn, ws, b)


def _m3(s1, acc, deg, wn2t, ws2t, b2):
  def body(s_ref, a_ref, d_ref, wn_ref, ws_ref, b_ref, vw_ref):
    h1 = jnp.maximum(s_ref[...] + (a_ref[0] + a_ref[1]) * _rdeg(d_ref), 0.0)
    vw_ref[:, :D] = jnp.dot(h1, wn_ref[...], preferred_element_type=F32)
    vw_ref[:, D:] = jnp.dot(h1, ws_ref[...], preferred_element_type=F32) + b_ref[...]

  return pl.pallas_call(
      body,
      out_shape=jax.ShapeDtypeStruct((N1A, 2 * D), F32),
  )(s1, acc, deg, wn2t, ws2t, b2)


def _m4(g, h0, wn2b, ws2b):
  def body(g_ref, h_ref, wn_ref, ws_ref, u_ref, s_ref):
    hb = h_ref[...]
    u_ref[...] = g_ref[:, :D] + jnp.dot(hb, wn_ref[...],
                                        preferred_element_type=F32)
    s_ref[...] = g_ref[:, D:] + jnp.dot(hb, ws_ref[...], preferred_element_type=F32)

  return pl.pallas_call(
      body,
      grid=(N0A // _RB,),
      in_specs=[
          pl.BlockSpec((_RB, 2 * D), lambda i: (i, 0)),
          pl.BlockSpec((_RB, D), lambda i: (i, 0)),
          pl.BlockSpec((D, D), lambda i: (0, 0)),
          pl.BlockSpec((D, D), lambda i: (0, 0)),
      ],
      out_specs=[
          pl.BlockSpec((_RB, D), lambda i: (i, 0)),
          pl.BlockSpec((_RB, D), lambda i: (i, 0)),
      ],
      out_shape=[
          jax.ShapeDtypeStruct((N0A, D), F32),
          jax.ShapeDtypeStruct((N0A, D), F32),
      ],
  )(g, h0, wn2b, ws2b)


def _m5(s2, acc, deg, wm, bm):
  def body(s_ref, a_ref, d_ref, wm_ref, bm_ref, z_ref):
    i = pl.program_id(0)
    h = jnp.maximum(s_ref[...] + (a_ref[0] + a_ref[1]) * _rdeg(d_ref), 0.0)
    z = jnp.dot(h, wm_ref[...], preferred_element_type=F32) + bm_ref[0]
    rows = i * _RB + lax.broadcasted_iota(I32, (_RB, 1), 0)
    z_ref[...] = jnp.where(rows < N0, z, -1e30)

  return pl.pallas_call(
      body,
      grid=(N0A // _RB,),
      in_specs=[
          pl.BlockSpec((_RB, D), lambda i: (i, 0)),
          pl.BlockSpec((2, _RB, D), lambda i: (0, i, 0)),
          pl.BlockSpec((_RB, 1), lambda i: (i, 0)),
          pl.BlockSpec((D, 1), lambda i: (0, 0)),
          pl.BlockSpec((1,), lambda i: (0,)),
      ],
      out_specs=pl.BlockSpec((_RB, 1), lambda i: (i, 0)),
      out_shape=jax.ShapeDtypeStruct((N0A, 1), F32),
  )(s2, acc, deg, wm, bm)


def _m6(z):
  def body(z_ref, o_ref):
    zv = z_ref[...]
    e = jnp.exp(zv - jnp.max(zv))
    o_ref[...] = e / jnp.sum(e)

  return pl.pallas_call(
      body,
      out_shape=jax.ShapeDtypeStruct((N0A // 128, 128), F32),
  )(z)


def kernel(x, e0_src, e0_dst, e1_src, e1_dst, down_dst,
           w_self0, w_neigh0, b0, w_self1, w_neigh1, b1,
           w_self2, w_neigh2, b2, w_mlp, b_mlp):
  p0 = E0P - E0
  pad0s = ((jnp.arange(p0, dtype=I32) * 13) % N0A)
  pad0d = N0 + (jnp.arange(p0, dtype=I32) % (N0A - N0))
  e0s = jnp.concatenate([e0_src.astype(I32), pad0s])
  e0d = jnp.concatenate([e0_dst.astype(I32), pad0d])
  p1 = E1P - E1
  pad1s = ((jnp.arange(p1, dtype=I32) * 13) % N1A)
  pad1d = N1 + (jnp.arange(p1, dtype=I32) % (N1A - N1))
  e1s = jnp.concatenate([e1_src.astype(I32), pad1s])
  e1d = jnp.concatenate([e1_dst.astype(I32), pad1d])
  pd = N0A - N0
  ddst = jnp.concatenate(
      [down_dst.astype(I32), N1 + (jnp.arange(pd, dtype=I32) % (N1A - N1))])
  idx0 = jnp.arange(N0A, dtype=I32)
  xp = jnp.pad(x, ((0, N0A - N0), (0, 0)))

  z128 = jnp.zeros((128, D), F32)
  z0a = jnp.zeros((N0A,), F32)
  z1a = jnp.zeros((N1A,), F32)

  deg0, pdeg, deg1 = _hist3_pass()(e0d, ddst, e1d, z0a, z1a)
  deg0c = (deg0[0] + deg0[1])[:, None]
  pdegc = (pdeg[0] + pdeg[1])[:, None]
  deg1c = (deg1[0] + deg1[1])[:, None]

  e0s2 = e0s.reshape(E0P // 128, 128)
  e0d2 = e0d.reshape(E0P // 128, 128)
  e1s2 = e1s.reshape(E1P // 128, 128)
  e1d2 = e1d.reshape(E1P // 128, 128)

  u0, s0 = _m0(xp, w_neigh0, w_self0, b0)
  acc0 = _seg_pass(N0A, 10240, 128, 80)(u0, e0s2, e0d2, z128)
  h0 = _m1(s0, acc0, deg0c)

  accd = _seg_pass(N1A, 320, 80, 4)(h0, idx0, ddst, z128)
  u1, s1 = _m2(accd, pdegc, w_neigh1, w_self1, b1)

  acc1 = _seg_pass(N1A, 1024, 128, 8)(u1, e1s2, e1d2, z128)
  vw = _m3(s1, acc1, deg1c, w_neigh2[:D], w_self2[:D], b2)

  g = _gather_pass(N0A, N1A, 2 * D, 80, 4)(vw, ddst)
  u2, s2 = _m4(g, h0, w_neigh2[D:], w_self2[D:])
  acc2 = _seg_pass(N0A, 10240, 128, 80)(u2, e0s2, e0d2, z128)

  z = _m5(s2, acc2, deg0c, w_mlp, b_mlp)
  sm = _m6(z.reshape(N0A // 128, 128))
  return sm.reshape(N0A, 1)[:N0]
